# baseline (device time: 16678 ns/iter reference)
import jax
import jax.numpy as jnp
from jax import lax
from jax.experimental import pallas as pl
from jax.experimental.pallas import tpu as pltpu

N_DEV = 16
BLK = 64


def kernel(x, w_mat):
    k_dim, m_per = x.shape
    n = w_mat.shape[1]

    def body(x_hbm_ref, w_hbm_ref, out_hbm_ref, xv_ref, w_ref, comm_ref,
             out_stage_ref, ready_sems, send_sems, recv_sems,
             xcopy_sem, wcopy_sem, ocopy_sem):
        my = lax.axis_index("i")

        xcopy = pltpu.make_async_copy(x_hbm_ref, xv_ref, xcopy_sem)
        xcopy.start()
        wcopy = pltpu.make_async_copy(w_hbm_ref, w_ref, wcopy_sem)
        wcopy.start()

        barrier_sem = pltpu.get_barrier_semaphore()
        pl.semaphore_signal(barrier_sem, inc=1)
        pl.semaphore_wait(barrier_sem, 1)

        for d in range(1, N_DEV):
            peer = lax.rem(my + d, N_DEV)
            pl.semaphore_signal(
                ready_sems.at[my], inc=1,
                device_id=(peer,), device_id_type=pl.DeviceIdType.MESH,
            )

        xcopy.wait()

        comm_ref[pl.ds(my * BLK, BLK), :] = xv_ref[pl.ds(my * BLK, BLK), :]

        rdmas = []
        for d in range(1, N_DEV):
            peer = lax.rem(my + d, N_DEV)
            pl.semaphore_wait(ready_sems.at[peer], 1)
            rdma = pltpu.make_async_remote_copy(
                src_ref=xv_ref.at[pl.ds(peer * BLK, BLK), :],
                dst_ref=comm_ref.at[pl.ds(my * BLK, BLK), :],
                send_sem=send_sems.at[d],
                recv_sem=recv_sems.at[d],
                device_id=(peer,),
                device_id_type=pl.DeviceIdType.MESH,
            )
            rdma.start()
            rdmas.append(rdma)

        for rdma in rdmas:
            rdma.wait_recv()

        x_rows = jnp.concatenate(
            [comm_ref[j * BLK:(j + 1) * BLK, :] for j in range(N_DEV)], axis=1
        )
        wcopy.wait()
        y = jnp.dot(x_rows, w_ref[:, :], preferred_element_type=jnp.float32)
        out_stage_ref[:, :] = y * jax.nn.sigmoid(y)

        ocopy = pltpu.make_async_copy(out_stage_ref, out_hbm_ref, ocopy_sem)
        ocopy.start()
        for rdma in rdmas:
            rdma.wait_send()
        ocopy.wait()

    return pl.pallas_call(
        body,
        out_shape=jax.ShapeDtypeStruct((BLK, n), jnp.float32),
        in_specs=[
            pl.BlockSpec(memory_space=pltpu.HBM),
            pl.BlockSpec(memory_space=pltpu.HBM),
        ],
        out_specs=pl.BlockSpec(memory_space=pltpu.HBM),
        scratch_shapes=[
            pltpu.VMEM((k_dim, m_per), jnp.float32),
            pltpu.VMEM((k_dim, n), jnp.float32),
            pltpu.VMEM((k_dim, m_per), jnp.float32),
            pltpu.VMEM((BLK, n), jnp.float32),
            pltpu.SemaphoreType.REGULAR((N_DEV,)),
            pltpu.SemaphoreType.DMA((N_DEV,)),
            pltpu.SemaphoreType.DMA((N_DEV,)),
            pltpu.SemaphoreType.DMA,
            pltpu.SemaphoreType.DMA,
            pltpu.SemaphoreType.DMA,
        ],
        compiler_params=pltpu.CompilerParams(collective_id=0),
    )(x, w_mat)
